# baseline (speedup 1.0000x reference)
import functools

import jax
import jax.numpy as jnp
from jax import lax
from jax.experimental import pallas as pl
from jax.experimental.pallas import tpu as pltpu
from jax.experimental.pallas import tpu_sc as plsc

N = 160000
NUM_ELEMENTS = 100
NUM_FEATURES = 64
M = 4
ROW = M * NUM_FEATURES
NTAB = NUM_ELEMENTS * NUM_ELEMENTS

NW = 32
CHUNK = 128
CPW = 40
NPAD = NW * CHUNK * CPW

_sc_mesh = plsc.VectorSubcoreMesh(core_axis_name="c", subcore_axis_name="s")


@functools.partial(
    pl.kernel,
    out_type=jax.ShapeDtypeStruct((NPAD // 2, 2 * NUM_FEATURES), jnp.float32),
    mesh=_sc_mesh,
    scratch_types=[
        pltpu.VMEM((CPW, CHUNK), jnp.int32),
        pltpu.VMEM((CPW, CHUNK), jnp.float32),
        pltpu.VMEM((CHUNK, ROW), jnp.float32),
        pltpu.VMEM((CHUNK, ROW), jnp.float32),
        pltpu.VMEM((CHUNK // 2, 2 * NUM_FEATURES), jnp.float32),
        pltpu.VMEM((CHUNK // 2, 2 * NUM_FEATURES), jnp.float32),
        pltpu.SemaphoreType.DMA,
        pltpu.SemaphoreType.DMA,
        pltpu.SemaphoreType.DMA,
        pltpu.SemaphoreType.DMA,
    ],
)
def _sc_shift(tab, idx, ld, out, idx_v, ld_v, buf_a, buf_b, sbuf_a, sbuf_b,
              sem_a, sem_b, wsem_a, wsem_b):
    wid = lax.axis_index("s") * 2 + lax.axis_index("c")
    obase = wid * (CPW * CHUNK // 2)
    pltpu.sync_copy(idx.at[wid], idx_v)
    pltpu.sync_copy(ld.at[wid], ld_v)

    def compute(buf, sbuf, jj):
        def cgroup(g, carry):
            lv = ld_v[jj, pl.ds(g * 16, 16)]
            for r16 in range(16):
                r = g * 16 + r16
                sc = lv[r16]
                half = (r16 % 2) * NUM_FEATURES
                for f in range(NUM_FEATURES // 16):
                    pv = buf[r, pl.ds(f * 16, 16)]
                    bv = buf[r, pl.ds(NUM_FEATURES + f * 16, 16)]
                    wv = buf[r, pl.ds(2 * NUM_FEATURES + f * 16, 16)]
                    sh = pv * jnp.exp(-jnp.exp(wv * sc - bv))
                    sbuf[r // 2, pl.ds(half + f * 16, 16)] = sh
            return carry

        lax.fori_loop(0, CHUNK // 16, cgroup, 0)

    pltpu.async_copy(tab.at[idx_v.at[0]], buf_a, sem_a)

    def body(t, carry):
        j0 = 2 * t
        j1 = j0 + 1
        pltpu.async_copy(tab.at[idx_v.at[j1]], buf_b, sem_b)
        pltpu.make_async_copy(tab.at[idx_v.at[j0]], buf_a, sem_a).wait()

        @pl.when(t > 0)
        def _():
            pltpu.make_async_copy(
                sbuf_a, out.at[pl.ds(obase, CHUNK // 2)], wsem_a).wait()

        compute(buf_a, sbuf_a, j0)

        @pl.when(t < CPW // 2 - 1)
        def _():
            pltpu.async_copy(tab.at[idx_v.at[j0 + 2]], buf_a, sem_a)

        pltpu.async_copy(
            sbuf_a, out.at[pl.ds(obase + j0 * (CHUNK // 2), CHUNK // 2)],
            wsem_a)

        pltpu.make_async_copy(tab.at[idx_v.at[j1]], buf_b, sem_b).wait()

        @pl.when(t > 0)
        def _():
            pltpu.make_async_copy(
                sbuf_b, out.at[pl.ds(obase, CHUNK // 2)], wsem_b).wait()

        compute(buf_b, sbuf_b, j1)
        pltpu.async_copy(
            sbuf_b, out.at[pl.ds(obase + j1 * (CHUNK // 2), CHUNK // 2)],
            wsem_b)
        return carry

    lax.fori_loop(0, CPW // 2, body, 0)
    pltpu.make_async_copy(sbuf_a, out.at[pl.ds(obase, CHUNK // 2)],
                          wsem_a).wait()
    pltpu.make_async_copy(sbuf_b, out.at[pl.ds(obase, CHUNK // 2)],
                          wsem_b).wait()


def _tc_pre(tl_ref, tw_ref, d_ref, b_ref, ld_ref):
    b_ref[...] = tw_ref[...] * jnp.log(tl_ref[...])
    ld_ref[...] = jnp.log(d_ref[...])


def _tc_add(x_ref, s_ref, o_ref):
    s = s_ref[...]
    r2 = s.shape[0]
    z = jnp.zeros((r2, ROW - NUM_FEATURES), jnp.float32)
    upd = jnp.concatenate(
        [s[:, :NUM_FEATURES], z, s[:, NUM_FEATURES:], z], axis=1)
    o_ref[...] = x_ref[...] + upd


def kernel(x, d, Z_i, Z_j, exp_prefactors, exp_lengthscales, exp_powers):
    idx = Z_i.astype(jnp.int32) * NUM_ELEMENTS + Z_j.astype(jnp.int32)
    idx = jnp.concatenate([idx, jnp.zeros((NPAD - N,), jnp.int32)])
    idx3 = idx.reshape(NW, CPW, CHUNK)
    tp = exp_prefactors.reshape(NTAB, NUM_FEATURES)
    tl = exp_lengthscales.reshape(NTAB, NUM_FEATURES)
    tw = exp_powers.reshape(NTAB, NUM_FEATURES)

    btab, ld2 = pl.pallas_call(
        _tc_pre,
        out_shape=[
            jax.ShapeDtypeStruct((NTAB, NUM_FEATURES), jnp.float32),
            jax.ShapeDtypeStruct((N // 128, 128), jnp.float32),
        ],
    )(tl, tw, d.reshape(N // 128, 128))

    tab = jnp.concatenate([tp, btab, tw, tw], axis=-1)
    ldp = jnp.concatenate([ld2.reshape(N), jnp.zeros((NPAD - N,), jnp.float32)])
    ld3 = ldp.reshape(NW, CPW, CHUNK)

    s = _sc_shift(tab, idx3, ld3)

    R2 = 400
    grid = (N // 2 // R2,)
    out2 = pl.pallas_call(
        _tc_add,
        grid=grid,
        in_specs=[
            pl.BlockSpec((R2, 2 * ROW), lambda i: (i, 0)),
            pl.BlockSpec((R2, 2 * NUM_FEATURES), lambda i: (i, 0)),
        ],
        out_specs=pl.BlockSpec((R2, 2 * ROW), lambda i: (i, 0)),
        out_shape=jax.ShapeDtypeStruct((N // 2, 2 * ROW), jnp.float32),
    )(x.reshape(N // 2, 2 * ROW), s)

    return out2.reshape(N, 1, M, NUM_FEATURES)


# speedup vs baseline: 1.3238x; 1.3238x over previous
import functools

import jax
import jax.numpy as jnp
from jax import lax
from jax.experimental import pallas as pl
from jax.experimental.pallas import tpu as pltpu
from jax.experimental.pallas import tpu_sc as plsc

N = 160000
NUM_ELEMENTS = 100
NUM_FEATURES = 64
M = 4
ROW = M * NUM_FEATURES
GROW = 3 * NUM_FEATURES
NTAB = NUM_ELEMENTS * NUM_ELEMENTS

NW = 32
CHUNK = 128
CPW = 40
NPAD = NW * CHUNK * CPW

_sc_mesh = plsc.VectorSubcoreMesh(core_axis_name="c", subcore_axis_name="s")


@functools.partial(
    pl.kernel,
    out_type=jax.ShapeDtypeStruct((NPAD, GROW), jnp.bfloat16),
    mesh=_sc_mesh,
    scratch_types=[
        pltpu.VMEM_SHARED((NTAB, GROW), jnp.bfloat16),
        pltpu.VMEM((CPW, CHUNK), jnp.int32),
        pltpu.VMEM((CHUNK, GROW), jnp.bfloat16),
        pltpu.VMEM((CHUNK, GROW), jnp.bfloat16),
        pltpu.SemaphoreType.DMA,
        pltpu.SemaphoreType.DMA,
    ],
    compiler_params=pltpu.CompilerParams(use_tc_tiling_on_sc=False),
)
def _sc_gather(tab, idx, out, stab, idx_v, buf_a, buf_b, sem_a, sem_b):
    sid = lax.axis_index("s")
    wid = sid * 2 + lax.axis_index("c")
    base = wid * (CPW * CHUNK)

    slab = 624
    pltpu.sync_copy(tab.at[pl.ds(sid * slab, slab)],
                    stab.at[pl.ds(sid * slab, slab)])

    @pl.when(sid == 0)
    def _():
        pltpu.sync_copy(tab.at[pl.ds(16 * slab, NTAB - 16 * slab)],
                        stab.at[pl.ds(16 * slab, NTAB - 16 * slab)])
    pltpu.sync_copy(idx.at[wid], idx_v)
    plsc.subcore_barrier()

    pltpu.async_copy(stab.at[idx_v.at[0]], buf_a, sem_a)

    def body(t, carry):
        j0 = 2 * t
        j1 = j0 + 1
        pltpu.async_copy(stab.at[idx_v.at[j1]], buf_b, sem_b)
        pltpu.make_async_copy(stab.at[idx_v.at[j0]], buf_a, sem_a).wait()
        pltpu.sync_copy(buf_a, out.at[pl.ds(base + j0 * CHUNK, CHUNK)])

        @pl.when(t < CPW // 2 - 1)
        def _():
            pltpu.async_copy(stab.at[idx_v.at[j0 + 2]], buf_a, sem_a)

        pltpu.make_async_copy(stab.at[idx_v.at[j1]], buf_b, sem_b).wait()
        pltpu.sync_copy(buf_b, out.at[pl.ds(base + j1 * CHUNK, CHUNK)])
        return carry

    lax.fori_loop(0, CPW // 2, body, 0)


def _tc_body(x_ref, d_ref, g_ref, o_ref):
    g = g_ref[...].astype(jnp.float32)
    p = g[:, 0:NUM_FEATURES]
    l = g[:, NUM_FEATURES:2 * NUM_FEATURES]
    w = g[:, 2 * NUM_FEATURES:3 * NUM_FEATURES]
    d = d_ref[...]
    shift = p * jnp.exp(-jnp.exp(w * jnp.log(d / l)))
    pad = jnp.zeros((shift.shape[0], ROW - NUM_FEATURES), jnp.float32)
    o_ref[...] = x_ref[...] + jnp.concatenate([shift, pad], axis=1)


def kernel(x, d, Z_i, Z_j, exp_prefactors, exp_lengthscales, exp_powers):
    idx = Z_i.astype(jnp.int32) * NUM_ELEMENTS + Z_j.astype(jnp.int32)
    idx = jnp.concatenate([idx, jnp.zeros((NPAD - N,), jnp.int32)])
    idx3 = idx.reshape(NW, CPW, CHUNK)
    tp = exp_prefactors.reshape(NTAB, NUM_FEATURES)
    tl = exp_lengthscales.reshape(NTAB, NUM_FEATURES)
    tw = exp_powers.reshape(NTAB, NUM_FEATURES)
    tab = jnp.concatenate([tp, tl, tw], axis=-1).astype(jnp.bfloat16)

    g = _sc_gather(tab, idx3)

    R = 800
    grid = (N // R,)
    out2 = pl.pallas_call(
        _tc_body,
        grid=grid,
        in_specs=[
            pl.BlockSpec((R, ROW), lambda i: (i, 0)),
            pl.BlockSpec((R, 1), lambda i: (i, 0)),
            pl.BlockSpec((R, GROW), lambda i: (i, 0)),
        ],
        out_specs=pl.BlockSpec((R, ROW), lambda i: (i, 0)),
        out_shape=jax.ShapeDtypeStruct((N, ROW), jnp.float32),
    )(x.reshape(N, ROW), d.reshape(N, 1), g)

    return out2.reshape(N, 1, M, NUM_FEATURES)


# speedup vs baseline: 2.8845x; 2.1790x over previous
import functools

import jax
import jax.numpy as jnp
from jax import lax
from jax.experimental import pallas as pl
from jax.experimental.pallas import tpu as pltpu
from jax.experimental.pallas import tpu_sc as plsc

N = 160000
NUM_ELEMENTS = 100
NUM_FEATURES = 64
M = 4
ROW = M * NUM_FEATURES
PK = 2 * NUM_FEATURES
NTAB = NUM_ELEMENTS * NUM_ELEMENTS

NW = 32
CHUNK = 128
CPW = 40
NPAD = NW * CHUNK * CPW

_sc_mesh = plsc.VectorSubcoreMesh(core_axis_name="c", subcore_axis_name="s")


@functools.partial(
    pl.kernel,
    out_type=jax.ShapeDtypeStruct((NPAD, PK), jnp.float32),
    mesh=_sc_mesh,
    scratch_types=[
        pltpu.VMEM_SHARED((NTAB, PK), jnp.float32),
        pltpu.VMEM((CPW, CHUNK), jnp.int32),
        pltpu.VMEM((CHUNK, PK), jnp.float32),
        pltpu.VMEM((CHUNK, PK), jnp.float32),
        pltpu.SemaphoreType.DMA,
        pltpu.SemaphoreType.DMA,
    ],
)
def _sc_gather(tab, idx, out, stab, idx_v, buf_a, buf_b, sem_a, sem_b):
    sid = lax.axis_index("s")
    wid = sid * 2 + lax.axis_index("c")
    base = wid * (CPW * CHUNK)

    slab = 624
    pltpu.sync_copy(tab.at[pl.ds(sid * slab, slab)],
                    stab.at[pl.ds(sid * slab, slab)])

    @pl.when(sid == 0)
    def _():
        pltpu.sync_copy(tab.at[pl.ds(16 * slab, NTAB - 16 * slab)],
                        stab.at[pl.ds(16 * slab, NTAB - 16 * slab)])

    pltpu.sync_copy(idx.at[wid], idx_v)
    plsc.subcore_barrier()

    pltpu.async_copy(stab.at[idx_v.at[0]], buf_a, sem_a)

    def body(t, carry):
        j0 = 2 * t
        j1 = j0 + 1
        pltpu.async_copy(stab.at[idx_v.at[j1]], buf_b, sem_b)
        pltpu.make_async_copy(stab.at[idx_v.at[j0]], buf_a, sem_a).wait()
        pltpu.sync_copy(buf_a, out.at[pl.ds(base + j0 * CHUNK, CHUNK)])

        @pl.when(t < CPW // 2 - 1)
        def _():
            pltpu.async_copy(stab.at[idx_v.at[j0 + 2]], buf_a, sem_a)

        pltpu.make_async_copy(stab.at[idx_v.at[j1]], buf_b, sem_b).wait()
        pltpu.sync_copy(buf_b, out.at[pl.ds(base + j1 * CHUNK, CHUNK)])
        return carry

    lax.fori_loop(0, CPW // 2, body, 0)


def _tc_body(x_ref, d_ref, g_ref, o_ref):
    gi = lax.bitcast_convert_type(g_ref[...], jnp.int32)
    lo = lax.bitcast_convert_type(jnp.left_shift(gi, 16), jnp.float32)
    hi = lax.bitcast_convert_type(
        jnp.bitwise_and(gi, jnp.int32(-65536)), jnp.float32)
    d = d_ref[...]
    s_lo = lo[:, 0:32] * jnp.exp(
        -jnp.exp(lo[:, 64:96] * jnp.log(d / lo[:, 32:64])))
    s_hi = hi[:, 0:32] * jnp.exp(
        -jnp.exp(hi[:, 64:96] * jnp.log(d / hi[:, 32:64])))
    s = jnp.concatenate([s_lo, s_hi], axis=1)
    ri = lax.broadcasted_iota(jnp.int32, (ROW, NUM_FEATURES), 0)
    ci = lax.broadcasted_iota(jnp.int32, (ROW, NUM_FEATURES), 1)
    sel = (ri == ci).astype(jnp.float32)
    upd = lax.dot_general(sel, s, (((1,), (1,)), ((), ())),
                          preferred_element_type=jnp.float32)
    o_ref[...] = x_ref[...] + upd


def _interleave(t):
    tr = t.reshape(NTAB, NUM_FEATURES)
    return jnp.stack([tr[:, :32], tr[:, 32:]], axis=-1).reshape(
        NTAB, NUM_FEATURES)


def kernel(x, d, Z_i, Z_j, exp_prefactors, exp_lengthscales, exp_powers):
    idx = Z_i.astype(jnp.int32) * NUM_ELEMENTS + Z_j.astype(jnp.int32)
    idx = jnp.concatenate([idx, jnp.zeros((NPAD - N,), jnp.int32)])
    idx3 = idx.reshape(NW, CPW, CHUNK)

    tabb = jnp.concatenate(
        [_interleave(exp_prefactors), _interleave(exp_lengthscales),
         _interleave(exp_powers), _interleave(exp_powers)],
        axis=-1).astype(jnp.bfloat16)
    tab = lax.bitcast_convert_type(
        tabb.reshape(NTAB, PK, 2), jnp.float32)

    g = _sc_gather(tab, idx3)

    C = 640
    grid = (N // C,)
    xt = x.reshape(N, ROW).T
    out_t = pl.pallas_call(
        _tc_body,
        grid=grid,
        in_specs=[
            pl.BlockSpec((ROW, C), lambda i: (0, i)),
            pl.BlockSpec((C, 1), lambda i: (i, 0)),
            pl.BlockSpec((C, PK), lambda i: (i, 0)),
        ],
        out_specs=pl.BlockSpec((ROW, C), lambda i: (0, i)),
        out_shape=jax.ShapeDtypeStruct((ROW, N), jnp.float32),
    )(xt, d.reshape(N, 1), g)

    return out_t.T.reshape(N, 1, M, NUM_FEATURES)


# speedup vs baseline: 3.4450x; 1.1943x over previous
import functools

import jax
import jax.numpy as jnp
from jax import lax
from jax.experimental import pallas as pl
from jax.experimental.pallas import tpu as pltpu
from jax.experimental.pallas import tpu_sc as plsc

N = 160000
NUM_ELEMENTS = 100
NUM_FEATURES = 64
M = 4
ROW = M * NUM_FEATURES
PK = 2 * NUM_FEATURES
NTAB = NUM_ELEMENTS * NUM_ELEMENTS

NW = 32
CHUNK = 128
CPW = 40
NPAD = NW * CHUNK * CPW

_sc_mesh = plsc.VectorSubcoreMesh(core_axis_name="c", subcore_axis_name="s")


@functools.partial(
    pl.kernel,
    out_type=jax.ShapeDtypeStruct((NPAD, PK), jnp.float32),
    mesh=_sc_mesh,
    scratch_types=[
        pltpu.VMEM_SHARED((NTAB, PK), jnp.float32),
        pltpu.VMEM((CPW, CHUNK), jnp.int32),
        pltpu.VMEM((CHUNK, PK), jnp.float32),
        pltpu.VMEM((CHUNK, PK), jnp.float32),
        pltpu.SemaphoreType.DMA,
        pltpu.SemaphoreType.DMA,
    ],
)
def _sc_gather(tab, idx, out, stab, idx_v, buf_a, buf_b, sem_a, sem_b):
    sid = lax.axis_index("s")
    wid = sid * 2 + lax.axis_index("c")
    base = wid * (CPW * CHUNK)

    slab = 624
    pltpu.sync_copy(tab.at[pl.ds(sid * slab, slab)],
                    stab.at[pl.ds(sid * slab, slab)])

    @pl.when(sid == 0)
    def _():
        pltpu.sync_copy(tab.at[pl.ds(16 * slab, NTAB - 16 * slab)],
                        stab.at[pl.ds(16 * slab, NTAB - 16 * slab)])

    pltpu.sync_copy(idx.at[wid], idx_v)
    plsc.subcore_barrier()

    pltpu.async_copy(stab.at[idx_v.at[0]], buf_a, sem_a)

    def body(t, carry):
        j0 = 2 * t
        j1 = j0 + 1
        pltpu.async_copy(stab.at[idx_v.at[j1]], buf_b, sem_b)
        pltpu.make_async_copy(stab.at[idx_v.at[j0]], buf_a, sem_a).wait()
        pltpu.sync_copy(buf_a, out.at[pl.ds(base + j0 * CHUNK, CHUNK)])

        @pl.when(t < CPW // 2 - 1)
        def _():
            pltpu.async_copy(stab.at[idx_v.at[j0 + 2]], buf_a, sem_a)

        pltpu.make_async_copy(stab.at[idx_v.at[j1]], buf_b, sem_b).wait()
        pltpu.sync_copy(buf_b, out.at[pl.ds(base + j1 * CHUNK, CHUNK)])
        return carry

    lax.fori_loop(0, CPW // 2, body, 0)


def _tc_pre(tl_ref, tw_ref, b_ref):
    b_ref[...] = tw_ref[...] * jnp.log(tl_ref[...])


def _tc_body(x_ref, d_ref, g_ref, o_ref):
    gi = lax.bitcast_convert_type(g_ref[...], jnp.int32)
    lo = lax.bitcast_convert_type(jnp.left_shift(gi, 16), jnp.float32)
    hi = lax.bitcast_convert_type(
        jnp.bitwise_and(gi, jnp.int32(-65536)), jnp.float32)
    ri = lax.broadcasted_iota(jnp.int32, (PK, PK), 0)
    ci = lax.broadcasted_iota(jnp.int32, (PK, PK), 1)
    eye = (ri == ci).astype(jnp.float32)
    lot = lax.dot_general(eye, lo, (((1,), (1,)), ((), ())),
                          preferred_element_type=jnp.float32)
    hit = lax.dot_general(eye, hi, (((1,), (1,)), ((), ())),
                          preferred_element_type=jnp.float32)
    logd = jnp.log(d_ref[...])
    s_lo = lot[0:32] * jnp.exp(-jnp.exp(lot[64:96] * logd - lot[32:64]))
    s_hi = hit[0:32] * jnp.exp(-jnp.exp(hit[64:96] * logd - hit[32:64]))
    c = s_lo.shape[1]
    upd = jnp.concatenate(
        [s_lo, s_hi, jnp.zeros((ROW - NUM_FEATURES, c), jnp.float32)], axis=0)
    o_ref[...] = x_ref[...] + upd


def _interleave(t):
    tr = t.reshape(NTAB, NUM_FEATURES)
    return jnp.stack([tr[:, :32], tr[:, 32:]], axis=-1).reshape(
        NTAB, NUM_FEATURES)


def kernel(x, d, Z_i, Z_j, exp_prefactors, exp_lengthscales, exp_powers):
    idx = Z_i.astype(jnp.int32) * NUM_ELEMENTS + Z_j.astype(jnp.int32)
    idx = jnp.concatenate([idx, jnp.zeros((NPAD - N,), jnp.int32)])
    idx3 = idx.reshape(NW, CPW, CHUNK)

    tp = exp_prefactors.reshape(NTAB, NUM_FEATURES)
    tl = exp_lengthscales.reshape(NTAB, NUM_FEATURES)
    tw = exp_powers.reshape(NTAB, NUM_FEATURES)
    btab = pl.pallas_call(
        _tc_pre,
        out_shape=jax.ShapeDtypeStruct((NTAB, NUM_FEATURES), jnp.float32),
    )(tl, tw)

    tabb = jnp.concatenate(
        [_interleave(tp), _interleave(btab), _interleave(tw),
         _interleave(tw)],
        axis=-1).astype(jnp.bfloat16)
    tab = lax.bitcast_convert_type(
        tabb.reshape(NTAB, PK, 2), jnp.float32)

    g = _sc_gather(tab, idx3)

    C = 640
    grid = (N // C,)
    xt = x.reshape(N, ROW).T
    out_t = pl.pallas_call(
        _tc_body,
        grid=grid,
        in_specs=[
            pl.BlockSpec((ROW, C), lambda i: (0, i)),
            pl.BlockSpec((1, C), lambda i: (0, i)),
            pl.BlockSpec((C, PK), lambda i: (i, 0)),
        ],
        out_specs=pl.BlockSpec((ROW, C), lambda i: (0, i)),
        out_shape=jax.ShapeDtypeStruct((ROW, N), jnp.float32),
    )(xt, d.reshape(1, N), g)

    return out_t.T.reshape(N, 1, M, NUM_FEATURES)


# speedup vs baseline: 4.3587x; 1.2652x over previous
import functools

import jax
import jax.numpy as jnp
from jax import lax
from jax.experimental import pallas as pl
from jax.experimental.pallas import tpu as pltpu
from jax.experimental.pallas import tpu_sc as plsc

N = 160000
NUM_ELEMENTS = 100
NUM_FEATURES = 64
M = 4
ROW = M * NUM_FEATURES
PK = 2 * NUM_FEATURES
NTAB = NUM_ELEMENTS * NUM_ELEMENTS

NW = 32
CHUNK = 128
CPW = 40
NPAD = NW * CHUNK * CPW

_sc_mesh = plsc.VectorSubcoreMesh(core_axis_name="c", subcore_axis_name="s")


@functools.partial(
    pl.kernel,
    out_type=jax.ShapeDtypeStruct((NPAD, PK), jnp.float32),
    mesh=_sc_mesh,
    scratch_types=[
        pltpu.VMEM_SHARED((NTAB, PK), jnp.float32),
        pltpu.VMEM((CPW, CHUNK), jnp.int32),
        pltpu.VMEM((CHUNK, PK), jnp.float32),
        pltpu.VMEM((CHUNK, PK), jnp.float32),
        pltpu.SemaphoreType.DMA,
        pltpu.SemaphoreType.DMA,
    ],
)
def _sc_gather(tab, idx, out, stab, idx_v, buf_a, buf_b, sem_a, sem_b):
    sid = lax.axis_index("s")
    wid = sid * 2 + lax.axis_index("c")
    base = wid * (CPW * CHUNK)

    slab = 624
    pltpu.sync_copy(tab.at[pl.ds(sid * slab, slab)],
                    stab.at[pl.ds(sid * slab, slab)])

    @pl.when(sid == 0)
    def _():
        pltpu.sync_copy(tab.at[pl.ds(16 * slab, NTAB - 16 * slab)],
                        stab.at[pl.ds(16 * slab, NTAB - 16 * slab)])

    pltpu.sync_copy(idx.at[wid], idx_v)
    plsc.subcore_barrier()

    pltpu.async_copy(stab.at[idx_v.at[0]], buf_a, sem_a)

    def body(t, carry):
        j0 = 2 * t
        j1 = j0 + 1
        pltpu.async_copy(stab.at[idx_v.at[j1]], buf_b, sem_b)
        pltpu.make_async_copy(stab.at[idx_v.at[j0]], buf_a, sem_a).wait()
        pltpu.sync_copy(buf_a, out.at[pl.ds(base + j0 * CHUNK, CHUNK)])

        @pl.when(t < CPW // 2 - 1)
        def _():
            pltpu.async_copy(stab.at[idx_v.at[j0 + 2]], buf_a, sem_a)

        pltpu.make_async_copy(stab.at[idx_v.at[j1]], buf_b, sem_b).wait()
        pltpu.sync_copy(buf_b, out.at[pl.ds(base + j1 * CHUNK, CHUNK)])
        return carry

    lax.fori_loop(0, CPW // 2, body, 0)


def _tc_pre(tl_ref, tw_ref, b_ref):
    b_ref[...] = tw_ref[...] * jnp.log(tl_ref[...])


def _tc_body(x_ref, d_ref, g_ref, o_ref):
    gi = lax.bitcast_convert_type(g_ref[...], jnp.int32)
    lo = lax.bitcast_convert_type(jnp.left_shift(gi, 16), jnp.float32)
    hi = lax.bitcast_convert_type(
        jnp.bitwise_and(gi, jnp.int32(-65536)), jnp.float32)
    ri = lax.broadcasted_iota(jnp.int32, (PK, PK), 0)
    ci = lax.broadcasted_iota(jnp.int32, (PK, PK), 1)
    eye = (ri == ci).astype(jnp.float32)
    lot = lax.dot_general(eye, lo, (((1,), (1,)), ((), ())),
                          preferred_element_type=jnp.float32)
    hit = lax.dot_general(eye, hi, (((1,), (1,)), ((), ())),
                          preferred_element_type=jnp.float32)
    logd = jnp.log(d_ref[...])
    s_lo = lot[0:32] * jnp.exp(-jnp.exp(lot[64:96] * logd - lot[32:64]))
    s_hi = hit[0:32] * jnp.exp(-jnp.exp(hit[64:96] * logd - hit[32:64]))
    c = s_lo.shape[1]
    upd = jnp.concatenate(
        [s_lo, s_hi, jnp.zeros((ROW - NUM_FEATURES, c), jnp.float32)], axis=0)
    o_ref[...] = x_ref[...] + upd


def _interleave(t):
    tr = t.reshape(NTAB, NUM_FEATURES)
    return jnp.stack([tr[:, :32], tr[:, 32:]], axis=-1).reshape(
        NTAB, NUM_FEATURES)


def kernel(x, d, Z_i, Z_j, exp_prefactors, exp_lengthscales, exp_powers):
    idx = Z_i.astype(jnp.int32) * NUM_ELEMENTS + Z_j.astype(jnp.int32)
    idx = jnp.concatenate([idx, jnp.zeros((NPAD - N,), jnp.int32)])
    idx3 = idx.reshape(NW, CPW, CHUNK)

    tp = exp_prefactors.reshape(NTAB, NUM_FEATURES)
    tl = exp_lengthscales.reshape(NTAB, NUM_FEATURES)
    tw = exp_powers.reshape(NTAB, NUM_FEATURES)
    btab = pl.pallas_call(
        _tc_pre,
        out_shape=jax.ShapeDtypeStruct((NTAB, NUM_FEATURES), jnp.float32),
    )(tl, tw)

    tabb = jnp.concatenate(
        [_interleave(tp), _interleave(btab), _interleave(tw),
         _interleave(tw)],
        axis=-1).astype(jnp.bfloat16)
    tab = lax.bitcast_convert_type(
        tabb.reshape(NTAB, PK, 2), jnp.float32)

    g = _sc_gather(tab, idx3)

    C = 1280
    grid = (N // C,)
    xt = x.reshape(N, ROW).T
    out_t = pl.pallas_call(
        _tc_body,
        grid=grid,
        in_specs=[
            pl.BlockSpec((ROW, C), lambda i: (0, i)),
            pl.BlockSpec((1, C), lambda i: (0, i)),
            pl.BlockSpec((C, PK), lambda i: (i, 0)),
        ],
        out_specs=pl.BlockSpec((ROW, C), lambda i: (0, i)),
        out_shape=jax.ShapeDtypeStruct((ROW, N), jnp.float32),
    )(xt, d.reshape(1, N), g)

    return out_t.T.reshape(N, 1, M, NUM_FEATURES)


# speedup vs baseline: 4.9674x; 1.1396x over previous
import functools

import jax
import jax.numpy as jnp
from jax import lax
from jax.experimental import pallas as pl
from jax.experimental.pallas import tpu as pltpu
from jax.experimental.pallas import tpu_sc as plsc

N = 160000
NUM_ELEMENTS = 100
NUM_FEATURES = 64
M = 4
ROW = M * NUM_FEATURES
PK = 2 * NUM_FEATURES
NTAB = NUM_ELEMENTS * NUM_ELEMENTS

NW = 32
CHUNK = 128
CPW = 40
NPAD = NW * CHUNK * CPW

_sc_mesh = plsc.VectorSubcoreMesh(core_axis_name="c", subcore_axis_name="s")


@functools.partial(
    pl.kernel,
    out_type=jax.ShapeDtypeStruct((NPAD, PK), jnp.float32),
    mesh=_sc_mesh,
    scratch_types=[
        pltpu.VMEM_SHARED((NTAB, PK), jnp.float32),
        pltpu.VMEM((CPW, CHUNK), jnp.int32),
        pltpu.VMEM((CHUNK, PK), jnp.float32),
        pltpu.VMEM((CHUNK, PK), jnp.float32),
        pltpu.SemaphoreType.DMA,
        pltpu.SemaphoreType.DMA,
    ],
)
def _sc_gather(tab, idx, out, stab, idx_v, buf_a, buf_b, sem_a, sem_b):
    sid = lax.axis_index("s")
    wid = sid * 2 + lax.axis_index("c")
    base = wid * (CPW * CHUNK)

    slab = 624
    pltpu.sync_copy(tab.at[pl.ds(sid * slab, slab)],
                    stab.at[pl.ds(sid * slab, slab)])

    @pl.when(sid == 0)
    def _():
        pltpu.sync_copy(tab.at[pl.ds(16 * slab, NTAB - 16 * slab)],
                        stab.at[pl.ds(16 * slab, NTAB - 16 * slab)])

    pltpu.sync_copy(idx.at[wid], idx_v)
    plsc.subcore_barrier()

    pltpu.async_copy(stab.at[idx_v.at[0]], buf_a, sem_a)

    def body(t, carry):
        j0 = 2 * t
        j1 = j0 + 1
        pltpu.async_copy(stab.at[idx_v.at[j1]], buf_b, sem_b)
        pltpu.make_async_copy(stab.at[idx_v.at[j0]], buf_a, sem_a).wait()
        pltpu.sync_copy(buf_a, out.at[pl.ds(base + j0 * CHUNK, CHUNK)])

        @pl.when(t < CPW // 2 - 1)
        def _():
            pltpu.async_copy(stab.at[idx_v.at[j0 + 2]], buf_a, sem_a)

        pltpu.make_async_copy(stab.at[idx_v.at[j1]], buf_b, sem_b).wait()
        pltpu.sync_copy(buf_b, out.at[pl.ds(base + j1 * CHUNK, CHUNK)])
        return carry

    lax.fori_loop(0, CPW // 2, body, 0)


def _tc_pre(tl_ref, tw_ref, b_ref):
    b_ref[...] = tw_ref[...] * jnp.log(tl_ref[...])


def _tc_body(x_ref, d_ref, g_ref, o_ref):
    gi = lax.bitcast_convert_type(g_ref[...], jnp.int32)
    lo = lax.bitcast_convert_type(jnp.left_shift(gi, 16), jnp.float32)
    hi = lax.bitcast_convert_type(
        jnp.bitwise_and(gi, jnp.int32(-65536)), jnp.float32)
    ri = lax.broadcasted_iota(jnp.int32, (PK, PK), 0)
    ci = lax.broadcasted_iota(jnp.int32, (PK, PK), 1)
    eye = (ri == ci).astype(jnp.float32)
    lot = lax.dot_general(eye, lo, (((1,), (1,)), ((), ())),
                          preferred_element_type=jnp.float32)
    hit = lax.dot_general(eye, hi, (((1,), (1,)), ((), ())),
                          preferred_element_type=jnp.float32)
    logd = jnp.log(d_ref[...])
    s_lo = lot[0:32] * jnp.exp(-jnp.exp(lot[64:96] * logd - lot[32:64]))
    s_hi = hit[0:32] * jnp.exp(-jnp.exp(hit[64:96] * logd - hit[32:64]))
    c = s_lo.shape[1]
    upd = jnp.concatenate(
        [s_lo, s_hi, jnp.zeros((ROW - NUM_FEATURES, c), jnp.float32)], axis=0)
    o_ref[...] = x_ref[...] + upd


def _interleave(t):
    tr = t.reshape(NTAB, NUM_FEATURES)
    return jnp.stack([tr[:, :32], tr[:, 32:]], axis=-1).reshape(
        NTAB, NUM_FEATURES)


def kernel(x, d, Z_i, Z_j, exp_prefactors, exp_lengthscales, exp_powers):
    idx = Z_i.astype(jnp.int32) * NUM_ELEMENTS + Z_j.astype(jnp.int32)
    idx = jnp.concatenate([idx, jnp.zeros((NPAD - N,), jnp.int32)])
    idx3 = idx.reshape(NW, CPW, CHUNK)

    tp = exp_prefactors.reshape(NTAB, NUM_FEATURES)
    tl = exp_lengthscales.reshape(NTAB, NUM_FEATURES)
    tw = exp_powers.reshape(NTAB, NUM_FEATURES)
    btab = pl.pallas_call(
        _tc_pre,
        out_shape=jax.ShapeDtypeStruct((NTAB, NUM_FEATURES), jnp.float32),
    )(tl, tw)

    tabb = jnp.concatenate(
        [_interleave(tp), _interleave(btab), _interleave(tw),
         _interleave(tw)],
        axis=-1).astype(jnp.bfloat16)
    tab = lax.bitcast_convert_type(
        tabb.reshape(NTAB, PK, 2), jnp.float32)

    g = _sc_gather(tab, idx3)

    C = 3200
    grid = (N // C,)
    xt = x.reshape(N, ROW).T
    out_t = pl.pallas_call(
        _tc_body,
        grid=grid,
        in_specs=[
            pl.BlockSpec((ROW, C), lambda i: (0, i)),
            pl.BlockSpec((1, C), lambda i: (0, i)),
            pl.BlockSpec((C, PK), lambda i: (i, 0)),
        ],
        out_specs=pl.BlockSpec((ROW, C), lambda i: (0, i)),
        out_shape=jax.ShapeDtypeStruct((ROW, N), jnp.float32),
    )(xt, d.reshape(1, N), g)

    return out_t.T.reshape(N, 1, M, NUM_FEATURES)


# speedup vs baseline: 5.0714x; 1.0209x over previous
import functools

import jax
import jax.numpy as jnp
from jax import lax
from jax.experimental import pallas as pl
from jax.experimental.pallas import tpu as pltpu
from jax.experimental.pallas import tpu_sc as plsc

N = 160000
NUM_ELEMENTS = 100
NUM_FEATURES = 64
M = 4
ROW = M * NUM_FEATURES
PK = 2 * NUM_FEATURES
NTAB = NUM_ELEMENTS * NUM_ELEMENTS

NW = 32
CHUNK = 128
CPW = 40
NPAD = NW * CHUNK * CPW

_sc_mesh = plsc.VectorSubcoreMesh(core_axis_name="c", subcore_axis_name="s")


@functools.partial(
    pl.kernel,
    out_type=jax.ShapeDtypeStruct((NPAD, PK), jnp.float32),
    mesh=_sc_mesh,
    scratch_types=[
        pltpu.VMEM_SHARED((NTAB, PK), jnp.float32),
        pltpu.VMEM((CPW, CHUNK), jnp.int32),
        pltpu.VMEM((CHUNK, PK), jnp.float32),
        pltpu.VMEM((CHUNK, PK), jnp.float32),
        pltpu.SemaphoreType.DMA,
        pltpu.SemaphoreType.DMA,
    ],
)
def _sc_gather(tab, idx, out, stab, idx_v, buf_a, buf_b, sem_a, sem_b):
    sid = lax.axis_index("s")
    wid = sid * 2 + lax.axis_index("c")
    base = wid * (CPW * CHUNK)

    slab = 624
    pltpu.sync_copy(tab.at[pl.ds(sid * slab, slab)],
                    stab.at[pl.ds(sid * slab, slab)])

    @pl.when(sid == 0)
    def _():
        pltpu.sync_copy(tab.at[pl.ds(16 * slab, NTAB - 16 * slab)],
                        stab.at[pl.ds(16 * slab, NTAB - 16 * slab)])

    pltpu.sync_copy(idx.at[wid], idx_v)
    plsc.subcore_barrier()

    pltpu.async_copy(stab.at[idx_v.at[0]], buf_a, sem_a)

    def body(t, carry):
        j0 = 2 * t
        j1 = j0 + 1
        pltpu.async_copy(stab.at[idx_v.at[j1]], buf_b, sem_b)
        pltpu.make_async_copy(stab.at[idx_v.at[j0]], buf_a, sem_a).wait()
        pltpu.sync_copy(buf_a, out.at[pl.ds(base + j0 * CHUNK, CHUNK)])

        @pl.when(t < CPW // 2 - 1)
        def _():
            pltpu.async_copy(stab.at[idx_v.at[j0 + 2]], buf_a, sem_a)

        pltpu.make_async_copy(stab.at[idx_v.at[j1]], buf_b, sem_b).wait()
        pltpu.sync_copy(buf_b, out.at[pl.ds(base + j1 * CHUNK, CHUNK)])
        return carry

    lax.fori_loop(0, CPW // 2, body, 0)


def _tc_pre(tl_ref, tw_ref, b_ref):
    b_ref[...] = tw_ref[...] * jnp.log(tl_ref[...])


def _tc_body(x_ref, d_ref, g_ref, o_ref):
    gi = lax.bitcast_convert_type(g_ref[...], jnp.int32)
    lo = lax.bitcast_convert_type(jnp.left_shift(gi, 16), jnp.float32)
    hi = lax.bitcast_convert_type(
        jnp.bitwise_and(gi, jnp.int32(-65536)), jnp.float32)
    ri = lax.broadcasted_iota(jnp.int32, (PK, PK), 0)
    ci = lax.broadcasted_iota(jnp.int32, (PK, PK), 1)
    eye = (ri == ci).astype(jnp.float32)
    lot = lax.dot_general(eye, lo, (((1,), (1,)), ((), ())),
                          preferred_element_type=jnp.float32)
    hit = lax.dot_general(eye, hi, (((1,), (1,)), ((), ())),
                          preferred_element_type=jnp.float32)
    logd = jnp.log(d_ref[...])
    s_lo = lot[0:32] * jnp.exp(-jnp.exp(lot[64:96] * logd - lot[32:64]))
    s_hi = hit[0:32] * jnp.exp(-jnp.exp(hit[64:96] * logd - hit[32:64]))
    c = s_lo.shape[1]
    upd = jnp.concatenate(
        [s_lo, s_hi, jnp.zeros((ROW - NUM_FEATURES, c), jnp.float32)], axis=0)
    o_ref[...] = x_ref[...] + upd


def _interleave(t):
    tr = t.reshape(NTAB, NUM_FEATURES)
    return jnp.stack([tr[:, :32], tr[:, 32:]], axis=-1).reshape(
        NTAB, NUM_FEATURES)


def kernel(x, d, Z_i, Z_j, exp_prefactors, exp_lengthscales, exp_powers):
    idx = Z_i.astype(jnp.int32) * NUM_ELEMENTS + Z_j.astype(jnp.int32)
    idx = jnp.concatenate([idx, jnp.zeros((NPAD - N,), jnp.int32)])
    idx3 = idx.reshape(NW, CPW, CHUNK)

    tp = exp_prefactors.reshape(NTAB, NUM_FEATURES)
    tl = exp_lengthscales.reshape(NTAB, NUM_FEATURES)
    tw = exp_powers.reshape(NTAB, NUM_FEATURES)
    btab = pl.pallas_call(
        _tc_pre,
        out_shape=jax.ShapeDtypeStruct((NTAB, NUM_FEATURES), jnp.float32),
    )(tl, tw)

    tabb = jnp.concatenate(
        [_interleave(tp), _interleave(btab), _interleave(tw),
         _interleave(tw)],
        axis=-1).astype(jnp.bfloat16)
    tab = lax.bitcast_convert_type(
        tabb.reshape(NTAB, PK, 2), jnp.float32)

    g = _sc_gather(tab, idx3)

    C = 6400
    grid = (N // C,)
    xt = x.reshape(N, ROW).T
    out_t = pl.pallas_call(
        _tc_body,
        grid=grid,
        in_specs=[
            pl.BlockSpec((ROW, C), lambda i: (0, i)),
            pl.BlockSpec((1, C), lambda i: (0, i)),
            pl.BlockSpec((C, PK), lambda i: (i, 0)),
        ],
        out_specs=pl.BlockSpec((ROW, C), lambda i: (0, i)),
        out_shape=jax.ShapeDtypeStruct((ROW, N), jnp.float32),
    )(xt, d.reshape(1, N), g)

    return out_t.T.reshape(N, 1, M, NUM_FEATURES)
